# Initial kernel scaffold; baseline (speedup 1.0000x reference)
import jax
import jax.numpy as jnp
from jax.experimental import pallas as pl


def kernel(x, tables, W1, b1, W2, b2, W3, b3):
    raise NotImplementedError("write your pallas kernel here")


# speedup vs baseline: 65.9826x; 65.9826x over previous
import functools

import numpy as np
import jax
import jax.numpy as jnp
from jax import lax
from jax.experimental import pallas as pl
from jax.experimental.pallas import tpu as pltpu
from jax.experimental.pallas import tpu_sc as plsc

_NUM_LEVELS = 16
_BASE_RES = 16
_MAX_RES = 2048
_LOG2_T = 19
_T = 2 ** _LOG2_T
_F = 2
_N = 1048576
_DH = 64
_HASH_K = np.uint32(2654435761)
_HMASK = np.uint32(_T - 1)

_GROWTH = np.exp((np.log(_MAX_RES) - np.log(_BASE_RES)) / (_NUM_LEVELS - 1))
_RES = [int(np.floor(_BASE_RES * (_GROWTH ** l))) for l in range(_NUM_LEVELS)]

_NC = 2
_NS = 16
_NW = _NC * _NS
_P = _N // _NW
_C = 2048
_NCH = _P // _C
_L = 16


def _encode_body(xc0, xc1, tabf0, tabf1, feats, xv0, xv1, i0, i1, i2, i3,
                 wv, g00, g01, g02, g03, g10, g11, g12, g13, facc, sem):
    idxs = (i0, i1, i2, i3)
    g0s = (g00, g01, g02, g03)
    g1s = (g10, g11, g12, g13)
    wid = lax.axis_index("s") * _NC + lax.axis_index("c")

    def chunk_body(ch, carry):
        base = wid * _P + ch * _C
        pltpu.sync_copy(xc0.at[pl.ds(base, _C)], xv0)
        pltpu.sync_copy(xc1.at[pl.ds(base, _C)], xv1)
        for l in range(_NUM_LEVELS):
            res = float(_RES[l])

            def cbody(i, c2):
                s = pl.ds(i * _L, _L)
                vx = xv0[s]
                vy = xv1[s]
                xs = vx * res
                ys = vy * res
                x0 = xs.astype(jnp.int32)
                y0 = ys.astype(jnp.int32)
                wx = xs - x0.astype(jnp.float32)
                wy = ys - y0.astype(jnp.float32)
                mx = 1.0 - wx
                my = 1.0 - wy
                u = x0.astype(jnp.uint32)
                u1 = u + np.uint32(1)
                a = y0.astype(jnp.uint32) * _HASH_K
                b = a + _HASH_K
                i0[s] = ((u ^ a) & _HMASK).astype(jnp.int32)
                i1[s] = ((u1 ^ a) & _HMASK).astype(jnp.int32)
                i2[s] = ((u ^ b) & _HMASK).astype(jnp.int32)
                i3[s] = ((u1 ^ b) & _HMASK).astype(jnp.int32)
                wv[0, s] = mx * my
                wv[1, s] = wx * my
                wv[2, s] = mx * wy
                wv[3, s] = wx * wy
                return c2

            lax.fori_loop(0, _C // _L, cbody, 0)

            tf0 = tabf0.at[pl.ds(l * _T, _T)]
            tf1 = tabf1.at[pl.ds(l * _T, _T)]
            copies = [
                pltpu.async_copy(tf0.at[idxs[c]], g0s[c], sem)
                for c in range(4)
            ] + [
                pltpu.async_copy(tf1.at[idxs[c]], g1s[c], sem)
                for c in range(4)
            ]
            for cp in copies:
                cp.wait()

            def abody(i, c2):
                s = pl.ds(i * _L, _L)
                acc0 = jnp.zeros((_L,), jnp.float32)
                acc1 = jnp.zeros((_L,), jnp.float32)
                for c in range(4):
                    w = wv[c, s]
                    acc0 = acc0 + g0s[c][s] * w
                    acc1 = acc1 + g1s[c][s] * w
                facc[2 * l, s] = acc0
                facc[2 * l + 1, s] = acc1
                return c2

            lax.fori_loop(0, _C // _L, abody, 0)

        pltpu.sync_copy(facc, feats.at[:, pl.ds(base, _C)])
        return carry

    lax.fori_loop(0, _NCH, chunk_body, 0)


@functools.partial(
    pl.kernel,
    out_type=jax.ShapeDtypeStruct((2 * _NUM_LEVELS, _N), jnp.float32),
    mesh=plsc.VectorSubcoreMesh(core_axis_name="c", subcore_axis_name="s"),
    scratch_types=[
        pltpu.VMEM((_C,), jnp.float32),
        pltpu.VMEM((_C,), jnp.float32),
        pltpu.VMEM((_C,), jnp.int32),
        pltpu.VMEM((_C,), jnp.int32),
        pltpu.VMEM((_C,), jnp.int32),
        pltpu.VMEM((_C,), jnp.int32),
        pltpu.VMEM((4, _C), jnp.float32),
        pltpu.VMEM((_C,), jnp.float32),
        pltpu.VMEM((_C,), jnp.float32),
        pltpu.VMEM((_C,), jnp.float32),
        pltpu.VMEM((_C,), jnp.float32),
        pltpu.VMEM((_C,), jnp.float32),
        pltpu.VMEM((_C,), jnp.float32),
        pltpu.VMEM((_C,), jnp.float32),
        pltpu.VMEM((_C,), jnp.float32),
        pltpu.VMEM((2 * _NUM_LEVELS, _C), jnp.float32),
        pltpu.SemaphoreType.DMA,
    ],
)
def _encode(xc0, xc1, tabf0, tabf1, feats, xv0, xv1, i0, i1, i2, i3, wv,
            g00, g01, g02, g03, g10, g11, g12, g13, facc, sem):
    _encode_body(xc0, xc1, tabf0, tabf1, feats, xv0, xv1, i0, i1, i2, i3,
                 wv, g00, g01, g02, g03, g10, g11, g12, g13, facc, sem)


_CB = 2048


def _mlp_body(f_ref, w1t_ref, b1_ref, w2t_ref, b2_ref, w3_ref, b3_ref, o_ref):
    f = f_ref[...]
    h = jnp.dot(w1t_ref[...], f, preferred_element_type=jnp.float32)
    h = jax.nn.gelu(h + b1_ref[...])
    h = jnp.dot(w2t_ref[...], h, preferred_element_type=jnp.float32)
    h = jax.nn.gelu(h + b2_ref[...])
    o = jnp.sum(h * w3_ref[...], axis=0, keepdims=True)
    o_ref[...] = o + b3_ref[...]


def _mlp(feats, w1t, b1, w2t, b2, w3, b3):
    din = 2 * _NUM_LEVELS
    grid = (_N // _CB,)
    return pl.pallas_call(
        _mlp_body,
        grid=grid,
        in_specs=[
            pl.BlockSpec((din, _CB), lambda i: (0, i)),
            pl.BlockSpec((_DH, din), lambda i: (0, 0)),
            pl.BlockSpec((_DH, 1), lambda i: (0, 0)),
            pl.BlockSpec((_DH, _DH), lambda i: (0, 0)),
            pl.BlockSpec((_DH, 1), lambda i: (0, 0)),
            pl.BlockSpec((_DH, 1), lambda i: (0, 0)),
            pl.BlockSpec((1, 1), lambda i: (0, 0)),
        ],
        out_specs=pl.BlockSpec((1, _CB), lambda i: (0, i)),
        out_shape=jax.ShapeDtypeStruct((1, _N), jnp.float32),
    )(feats, w1t, b1, w2t, b2, w3, b3)


def kernel(x, tables, W1, b1, W2, b2, W3, b3):
    xc0 = x[:, 0]
    xc1 = x[:, 1]
    tabf0 = tables[:, :, 0].reshape(-1)
    tabf1 = tables[:, :, 1].reshape(-1)
    feats = _encode(xc0, xc1, tabf0, tabf1)
    out = _mlp(
        feats,
        W1.T,
        b1.reshape(_DH, 1),
        W2.T,
        b2.reshape(_DH, 1),
        W3,
        b3.reshape(1, 1),
    )
    return out.reshape(_N, 1)


# speedup vs baseline: 66.1449x; 1.0025x over previous
import functools

import numpy as np
import jax
import jax.numpy as jnp
from jax import lax
from jax.experimental import pallas as pl
from jax.experimental.pallas import tpu as pltpu
from jax.experimental.pallas import tpu_sc as plsc

_NUM_LEVELS = 16
_BASE_RES = 16
_MAX_RES = 2048
_LOG2_T = 19
_T = 2 ** _LOG2_T
_F = 2
_N = 1048576
_DH = 64
_HASH_K = np.uint32(2654435761)
_HMASK = np.uint32(_T - 1)

_GROWTH = np.exp((np.log(_MAX_RES) - np.log(_BASE_RES)) / (_NUM_LEVELS - 1))
_RES = [int(np.floor(_BASE_RES * (_GROWTH ** l))) for l in range(_NUM_LEVELS)]

_NC = 2
_NS = 16
_NW = _NC * _NS
_P = _N // _NW
_C = 2048
_NCH = _P // _C
_L = 16


def _encode_body(xc0, xc1, tab0, tab1, feats, xv0, xv1, i0, i1, i2, i3,
                 w0, w1, w2, w3,
                 r0a, r1a, r2a, r3a, r0b, r1b, r2b, r3b, facc, sem):
    idxs = (i0, i1, i2, i3)
    ws = (w0, w1, w2, w3)
    rowsa = (r0a, r1a, r2a, r3a)
    rowsb = (r0b, r1b, r2b, r3b)
    wid = lax.axis_index("s") * _NC + lax.axis_index("c")

    def chunk_body(ch, carry):
        base = wid * _P + ch * _C
        pltpu.sync_copy(xc0.at[pl.ds(base, _C)], xv0)
        pltpu.sync_copy(xc1.at[pl.ds(base, _C)], xv1)
        for l in range(_NUM_LEVELS):
            res = float(_RES[l])
            off = np.int32(l * _T)

            def cbody(i, c2):
                s = pl.ds(i * _L, _L)
                vx = xv0[s]
                vy = xv1[s]
                xs = vx * res
                ys = vy * res
                x0 = xs.astype(jnp.int32)
                y0 = ys.astype(jnp.int32)
                wx = xs - x0.astype(jnp.float32)
                wy = ys - y0.astype(jnp.float32)
                mx = 1.0 - wx
                my = 1.0 - wy
                u = x0.astype(jnp.uint32)
                u1 = u + np.uint32(1)
                a = y0.astype(jnp.uint32) * _HASH_K
                b = a + _HASH_K
                i0[s] = ((u ^ a) & _HMASK).astype(jnp.int32) + off
                i1[s] = ((u1 ^ a) & _HMASK).astype(jnp.int32) + off
                i2[s] = ((u ^ b) & _HMASK).astype(jnp.int32) + off
                i3[s] = ((u1 ^ b) & _HMASK).astype(jnp.int32) + off
                w0[s] = mx * my
                w1[s] = wx * my
                w2[s] = mx * wy
                w3[s] = wx * wy
                return c2

            lax.fori_loop(0, _C // _L, cbody, 0)

            copies = [
                pltpu.async_copy(tab0.at[idxs[c]], rowsa[c], sem)
                for c in range(4)
            ] + [
                pltpu.async_copy(tab1.at[idxs[c]], rowsb[c], sem)
                for c in range(4)
            ]
            for cp in copies:
                cp.wait()

            def abody(i, c2):
                s = pl.ds(i * _L, _L)
                acca = jnp.zeros((_L,), jnp.float32)
                accb = jnp.zeros((_L,), jnp.float32)
                for c in range(4):
                    wv = ws[c][s]
                    acca = acca + rowsa[c][s] * wv
                    accb = accb + rowsb[c][s] * wv
                facc[2 * l, s] = acca
                facc[2 * l + 1, s] = accb
                return c2

            lax.fori_loop(0, _C // _L, abody, 0)

        pltpu.sync_copy(facc, feats.at[:, pl.ds(base, _C)])
        return carry

    lax.fori_loop(0, _NCH, chunk_body, 0)


@functools.partial(
    pl.kernel,
    out_type=jax.ShapeDtypeStruct((2 * _NUM_LEVELS, _N), jnp.float32),
    mesh=plsc.VectorSubcoreMesh(core_axis_name="c", subcore_axis_name="s"),
    scratch_types=[
        pltpu.VMEM((_C,), jnp.float32),
        pltpu.VMEM((_C,), jnp.float32),
        pltpu.VMEM((_C,), jnp.int32),
        pltpu.VMEM((_C,), jnp.int32),
        pltpu.VMEM((_C,), jnp.int32),
        pltpu.VMEM((_C,), jnp.int32),
        pltpu.VMEM((_C,), jnp.float32),
        pltpu.VMEM((_C,), jnp.float32),
        pltpu.VMEM((_C,), jnp.float32),
        pltpu.VMEM((_C,), jnp.float32),
        pltpu.VMEM((_C,), jnp.float32),
        pltpu.VMEM((_C,), jnp.float32),
        pltpu.VMEM((_C,), jnp.float32),
        pltpu.VMEM((_C,), jnp.float32),
        pltpu.VMEM((_C,), jnp.float32),
        pltpu.VMEM((_C,), jnp.float32),
        pltpu.VMEM((_C,), jnp.float32),
        pltpu.VMEM((_C,), jnp.float32),
        pltpu.VMEM((2 * _NUM_LEVELS, _C), jnp.float32),
        pltpu.SemaphoreType.DMA,
    ],
)
def _encode(xc0, xc1, tab0, tab1, feats, xv0, xv1, i0, i1, i2, i3,
            w0, w1, w2, w3,
            r0a, r1a, r2a, r3a, r0b, r1b, r2b, r3b, facc, sem):
    _encode_body(xc0, xc1, tab0, tab1, feats, xv0, xv1, i0, i1, i2, i3,
                 w0, w1, w2, w3,
                 r0a, r1a, r2a, r3a, r0b, r1b, r2b, r3b, facc, sem)


_CB = 2048


def _mlp_body(f_ref, w1t_ref, b1_ref, w2t_ref, b2_ref, w3_ref, b3_ref, o_ref):
    f = f_ref[...]
    h = jnp.dot(w1t_ref[...], f, preferred_element_type=jnp.float32)
    h = jax.nn.gelu(h + b1_ref[...])
    h = jnp.dot(w2t_ref[...], h, preferred_element_type=jnp.float32)
    h = jax.nn.gelu(h + b2_ref[...])
    o = jnp.sum(h * w3_ref[...], axis=0, keepdims=True)
    o_ref[...] = o + b3_ref[...]


def _mlp(feats, w1t, b1, w2t, b2, w3, b3):
    din = 2 * _NUM_LEVELS
    grid = (_N // _CB,)
    return pl.pallas_call(
        _mlp_body,
        grid=grid,
        in_specs=[
            pl.BlockSpec((din, _CB), lambda i: (0, i)),
            pl.BlockSpec((_DH, din), lambda i: (0, 0)),
            pl.BlockSpec((_DH, 1), lambda i: (0, 0)),
            pl.BlockSpec((_DH, _DH), lambda i: (0, 0)),
            pl.BlockSpec((_DH, 1), lambda i: (0, 0)),
            pl.BlockSpec((_DH, 1), lambda i: (0, 0)),
            pl.BlockSpec((1, 1), lambda i: (0, 0)),
        ],
        out_specs=pl.BlockSpec((1, _CB), lambda i: (0, i)),
        out_shape=jax.ShapeDtypeStruct((1, _N), jnp.float32),
    )(feats, w1t, b1, w2t, b2, w3, b3)


def kernel(x, tables, W1, b1, W2, b2, W3, b3):
    xc0 = x[:, 0]
    xc1 = x[:, 1]
    tabf = tables.reshape(_NUM_LEVELS * _T, _F)
    tab0 = tabf[:, 0]
    tab1 = tabf[:, 1]
    feats = _encode(xc0, xc1, tab0, tab1)
    out = _mlp(
        feats,
        W1.T,
        b1.reshape(_DH, 1),
        W2.T,
        b2.reshape(_DH, 1),
        W3,
        b3.reshape(1, 1),
    )
    return out.reshape(_N, 1)


# speedup vs baseline: 70.9094x; 1.0720x over previous
import functools

import numpy as np
import jax
import jax.numpy as jnp
from jax import lax
from jax.experimental import pallas as pl
from jax.experimental.pallas import tpu as pltpu
from jax.experimental.pallas import tpu_sc as plsc

_NUM_LEVELS = 16
_BASE_RES = 16
_MAX_RES = 2048
_LOG2_T = 19
_T = 2 ** _LOG2_T
_F = 2
_N = 1048576
_DH = 64
_HASH_K = np.uint32(2654435761)
_HMASK = np.uint32(_T - 1)

_GROWTH = np.exp((np.log(_MAX_RES) - np.log(_BASE_RES)) / (_NUM_LEVELS - 1))
_RES = [int(np.floor(_BASE_RES * (_GROWTH ** l))) for l in range(_NUM_LEVELS)]

_NC = 2
_NS = 16
_NW = _NC * _NS
_P = _N // _NW
_C = 1024
_NCH = _P // _C
_L = 16


def _encode_body(xc0, xc1, tab0, tab1, feats, xv0, xv1,
                 ia0, ia1, wa0, wa1, ra0, ra1, rb0, rb1, facc, sem0, sem1):
    ias = (ia0, ia1)
    was = (wa0, wa1)
    ras = (ra0, ra1)
    rbs = (rb0, rb1)
    sems = (sem0, sem1)
    wid = lax.axis_index("s") * _NC + lax.axis_index("c")

    def cbody_level(l, slot):
        res = float(_RES[l])
        off = np.int32(l * _T)
        ia = ias[slot]
        wa = was[slot]

        def cbody(i, c2):
            s = i * _L
            vx = xv0[pl.ds(s, _L)]
            vy = xv1[pl.ds(s, _L)]
            xs = vx * res
            ys = vy * res
            x0 = xs.astype(jnp.int32)
            y0 = ys.astype(jnp.int32)
            wx = xs - x0.astype(jnp.float32)
            wy = ys - y0.astype(jnp.float32)
            mx = 1.0 - wx
            my = 1.0 - wy
            u = x0.astype(jnp.uint32)
            u1 = u + np.uint32(1)
            a = y0.astype(jnp.uint32) * _HASH_K
            b = a + _HASH_K
            ia[pl.ds(s, _L)] = ((u ^ a) & _HMASK).astype(jnp.int32) + off
            ia[pl.ds(_C + s, _L)] = ((u1 ^ a) & _HMASK).astype(jnp.int32) + off
            ia[pl.ds(2 * _C + s, _L)] = ((u ^ b) & _HMASK).astype(jnp.int32) + off
            ia[pl.ds(3 * _C + s, _L)] = ((u1 ^ b) & _HMASK).astype(jnp.int32) + off
            wa[pl.ds(s, _L)] = mx * my
            wa[pl.ds(_C + s, _L)] = wx * my
            wa[pl.ds(2 * _C + s, _L)] = mx * wy
            wa[pl.ds(3 * _C + s, _L)] = wx * wy
            return c2

        lax.fori_loop(0, _C // _L, cbody, 0)

    def issue_level(slot):
        return [
            pltpu.async_copy(tab0.at[ias[slot]], ras[slot], sems[slot]),
            pltpu.async_copy(tab1.at[ias[slot]], rbs[slot], sems[slot]),
        ]

    def abody_level(l, slot):
        wa = was[slot]
        ra = ras[slot]
        rb = rbs[slot]

        def abody(i, c2):
            s = i * _L
            acca = jnp.zeros((_L,), jnp.float32)
            accb = jnp.zeros((_L,), jnp.float32)
            for c in range(4):
                sc_ = pl.ds(c * _C + s, _L)
                wv = wa[sc_]
                acca = acca + ra[sc_] * wv
                accb = accb + rb[sc_] * wv
            facc[2 * l, pl.ds(s, _L)] = acca
            facc[2 * l + 1, pl.ds(s, _L)] = accb
            return c2

        lax.fori_loop(0, _C // _L, abody, 0)

    def chunk_body(ch, carry):
        base = wid * _P + ch * _C
        pltpu.sync_copy(xc0.at[pl.ds(base, _C)], xv0)
        pltpu.sync_copy(xc1.at[pl.ds(base, _C)], xv1)
        cbody_level(0, 0)
        copies = issue_level(0)
        for l in range(_NUM_LEVELS):
            slot = l % 2
            nxt = (l + 1) % 2
            if l + 1 < _NUM_LEVELS:
                cbody_level(l + 1, nxt)
                next_copies = issue_level(nxt)
            else:
                next_copies = None
            for cp in copies:
                cp.wait()
            abody_level(l, slot)
            copies = next_copies
        pltpu.sync_copy(facc, feats.at[:, pl.ds(base, _C)])
        return carry

    lax.fori_loop(0, _NCH, chunk_body, 0)


@functools.partial(
    pl.kernel,
    out_type=jax.ShapeDtypeStruct((2 * _NUM_LEVELS, _N), jnp.float32),
    mesh=plsc.VectorSubcoreMesh(core_axis_name="c", subcore_axis_name="s"),
    scratch_types=[
        pltpu.VMEM((_C,), jnp.float32),
        pltpu.VMEM((_C,), jnp.float32),
        pltpu.VMEM((4 * _C,), jnp.int32),
        pltpu.VMEM((4 * _C,), jnp.int32),
        pltpu.VMEM((4 * _C,), jnp.float32),
        pltpu.VMEM((4 * _C,), jnp.float32),
        pltpu.VMEM((4 * _C,), jnp.float32),
        pltpu.VMEM((4 * _C,), jnp.float32),
        pltpu.VMEM((4 * _C,), jnp.float32),
        pltpu.VMEM((4 * _C,), jnp.float32),
        pltpu.VMEM((2 * _NUM_LEVELS, _C), jnp.float32),
        pltpu.SemaphoreType.DMA,
        pltpu.SemaphoreType.DMA,
    ],
)
def _encode(xc0, xc1, tab0, tab1, feats, xv0, xv1,
            ia0, ia1, wa0, wa1, ra0, ra1, rb0, rb1, facc, sem0, sem1):
    _encode_body(xc0, xc1, tab0, tab1, feats, xv0, xv1,
                 ia0, ia1, wa0, wa1, ra0, ra1, rb0, rb1, facc, sem0, sem1)


_CB = 2048


def _mlp_body(f_ref, w1t_ref, b1_ref, w2t_ref, b2_ref, w3_ref, b3_ref, o_ref):
    f = f_ref[...]
    h = jnp.dot(w1t_ref[...], f, preferred_element_type=jnp.float32)
    h = jax.nn.gelu(h + b1_ref[...])
    h = jnp.dot(w2t_ref[...], h, preferred_element_type=jnp.float32)
    h = jax.nn.gelu(h + b2_ref[...])
    o = jnp.sum(h * w3_ref[...], axis=0, keepdims=True)
    o_ref[...] = o + b3_ref[...]


def _mlp(feats, w1t, b1, w2t, b2, w3, b3):
    din = 2 * _NUM_LEVELS
    grid = (_N // _CB,)
    return pl.pallas_call(
        _mlp_body,
        grid=grid,
        in_specs=[
            pl.BlockSpec((din, _CB), lambda i: (0, i)),
            pl.BlockSpec((_DH, din), lambda i: (0, 0)),
            pl.BlockSpec((_DH, 1), lambda i: (0, 0)),
            pl.BlockSpec((_DH, _DH), lambda i: (0, 0)),
            pl.BlockSpec((_DH, 1), lambda i: (0, 0)),
            pl.BlockSpec((_DH, 1), lambda i: (0, 0)),
            pl.BlockSpec((1, 1), lambda i: (0, 0)),
        ],
        out_specs=pl.BlockSpec((1, _CB), lambda i: (0, i)),
        out_shape=jax.ShapeDtypeStruct((1, _N), jnp.float32),
    )(feats, w1t, b1, w2t, b2, w3, b3)


def kernel(x, tables, W1, b1, W2, b2, W3, b3):
    xc0 = x[:, 0]
    xc1 = x[:, 1]
    tabf = tables.reshape(_NUM_LEVELS * _T, _F)
    tab0 = tabf[:, 0]
    tab1 = tabf[:, 1]
    feats = _encode(xc0, xc1, tab0, tab1)
    out = _mlp(
        feats,
        W1.T,
        b1.reshape(_DH, 1),
        W2.T,
        b2.reshape(_DH, 1),
        W3,
        b3.reshape(1, 1),
    )
    return out.reshape(_N, 1)


# speedup vs baseline: 126.2618x; 1.7806x over previous
import functools

import numpy as np
import jax
import jax.numpy as jnp
from jax import lax
from jax.experimental import pallas as pl
from jax.experimental.pallas import tpu as pltpu
from jax.experimental.pallas import tpu_sc as plsc

_NUM_LEVELS = 16
_BASE_RES = 16
_MAX_RES = 2048
_LOG2_T = 19
_T = 2 ** _LOG2_T
_F = 2
_N = 1048576
_DH = 64
_HASH_K = np.uint32(2654435761)
_HMASK = np.uint32(_T - 1)

_GROWTH = np.exp((np.log(_MAX_RES) - np.log(_BASE_RES)) / (_NUM_LEVELS - 1))
_RES = [int(np.floor(_BASE_RES * (_GROWTH ** l))) for l in range(_NUM_LEVELS)]

_NC = 2
_NS = 16
_NW = _NC * _NS
_P = _N // _NW
_C = 1024
_NCH = _P // _C
_L = 16

_NLOC = 6
_LW = [_RES[l] + 1 for l in range(_NLOC)]
_LSZ = [((w * w + 7) // 8) * 8 for w in _LW]
_LBASE = [int(np.sum(_LSZ[:l])) for l in range(_NLOC)]
_LTOT = int(np.sum(_LSZ))


def _local_index_table():
    li = np.zeros((_LTOT,), np.int32)
    for l in range(_NLOC):
        w = _LW[l]
        y, x = np.mgrid[0:w, 0:w].astype(np.uint32)
        h = ((x ^ (y * _HASH_K)) & _HMASK).astype(np.int32) + l * _T
        li[_LBASE[l]:_LBASE[l] + w * w] = h.ravel()
    return li


_LIDX = _local_index_table()


def _encode_body(xc0, xc1, tab0, tab1, lidx, feats, xv0, xv1,
                 ia0, ia1, wa0, wa1, ra0, ra1, rb0, rb1,
                 lidxv, ltv, lsh0, lsh1, facc, sem0, sem1):
    ias = (ia0, ia1)
    was = (wa0, wa1)
    ras = (ra0, ra1)
    rbs = (rb0, rb1)
    sems = (sem0, sem1)
    wid = lax.axis_index("s") * _NC + lax.axis_index("c")

    pltpu.sync_copy(lidx, lidxv)
    pltpu.async_copy(tab0.at[lidxv], ltv, sem0).wait()
    pltpu.sync_copy(ltv, lsh0)
    pltpu.async_copy(tab1.at[lidxv], ltv, sem0).wait()
    pltpu.sync_copy(ltv, lsh1)

    def cbody_level(l, slot):
        res = float(_RES[l])
        ia = ias[slot]
        wa = was[slot]
        local = l < _NLOC
        if local:
            lw = np.uint32(_LW[l])
            lbase = np.uint32(_LBASE[l])
        else:
            off = np.int32(l * _T)

        def cbody(i, c2):
            s = i * _L
            vx = xv0[pl.ds(s, _L)]
            vy = xv1[pl.ds(s, _L)]
            xs = vx * res
            ys = vy * res
            x0 = xs.astype(jnp.int32)
            y0 = ys.astype(jnp.int32)
            wx = xs - x0.astype(jnp.float32)
            wy = ys - y0.astype(jnp.float32)
            mx = 1.0 - wx
            my = 1.0 - wy
            u = x0.astype(jnp.uint32)
            yu = y0.astype(jnp.uint32)
            if local:
                j0 = yu * lw + u + lbase
                ia[pl.ds(s, _L)] = j0.astype(jnp.int32)
                ia[pl.ds(_C + s, _L)] = (j0 + np.uint32(1)).astype(jnp.int32)
                j2 = j0 + lw
                ia[pl.ds(2 * _C + s, _L)] = j2.astype(jnp.int32)
                ia[pl.ds(3 * _C + s, _L)] = (j2 + np.uint32(1)).astype(jnp.int32)
            else:
                u1 = u + np.uint32(1)
                a = yu * _HASH_K
                b = a + _HASH_K
                ia[pl.ds(s, _L)] = ((u ^ a) & _HMASK).astype(jnp.int32) + off
                ia[pl.ds(_C + s, _L)] = ((u1 ^ a) & _HMASK).astype(jnp.int32) + off
                ia[pl.ds(2 * _C + s, _L)] = ((u ^ b) & _HMASK).astype(jnp.int32) + off
                ia[pl.ds(3 * _C + s, _L)] = ((u1 ^ b) & _HMASK).astype(jnp.int32) + off
            wa[pl.ds(s, _L)] = mx * my
            wa[pl.ds(_C + s, _L)] = wx * my
            wa[pl.ds(2 * _C + s, _L)] = mx * wy
            wa[pl.ds(3 * _C + s, _L)] = wx * wy
            return c2

        lax.fori_loop(0, _C // _L, cbody, 0)

    def issue_level(l, slot):
        if l < _NLOC:
            src0, src1 = lsh0, lsh1
        else:
            src0, src1 = tab0, tab1
        return [
            pltpu.async_copy(src0.at[ias[slot]], ras[slot], sems[slot]),
            pltpu.async_copy(src1.at[ias[slot]], rbs[slot], sems[slot]),
        ]

    def abody_level(l, slot):
        wa = was[slot]
        ra = ras[slot]
        rb = rbs[slot]

        def abody(i, c2):
            s = i * _L
            acca = jnp.zeros((_L,), jnp.float32)
            accb = jnp.zeros((_L,), jnp.float32)
            for c in range(4):
                sc_ = pl.ds(c * _C + s, _L)
                wv = wa[sc_]
                acca = acca + ra[sc_] * wv
                accb = accb + rb[sc_] * wv
            facc[2 * l, pl.ds(s, _L)] = acca
            facc[2 * l + 1, pl.ds(s, _L)] = accb
            return c2

        lax.fori_loop(0, _C // _L, abody, 0)

    def chunk_body(ch, carry):
        base = wid * _P + ch * _C
        pltpu.sync_copy(xc0.at[pl.ds(base, _C)], xv0)
        pltpu.sync_copy(xc1.at[pl.ds(base, _C)], xv1)
        cbody_level(0, 0)
        copies = issue_level(0, 0)
        for l in range(_NUM_LEVELS):
            slot = l % 2
            nxt = (l + 1) % 2
            if l + 1 < _NUM_LEVELS:
                cbody_level(l + 1, nxt)
                next_copies = issue_level(l + 1, nxt)
            else:
                next_copies = None
            for cp in copies:
                cp.wait()
            abody_level(l, slot)
            copies = next_copies
        pltpu.sync_copy(facc, feats.at[:, pl.ds(base, _C)])
        return carry

    lax.fori_loop(0, _NCH, chunk_body, 0)


@functools.partial(
    pl.kernel,
    out_type=jax.ShapeDtypeStruct((2 * _NUM_LEVELS, _N), jnp.float32),
    mesh=plsc.VectorSubcoreMesh(core_axis_name="c", subcore_axis_name="s"),
    scratch_types=[
        pltpu.VMEM((_C,), jnp.float32),
        pltpu.VMEM((_C,), jnp.float32),
        pltpu.VMEM((4 * _C,), jnp.int32),
        pltpu.VMEM((4 * _C,), jnp.int32),
        pltpu.VMEM((4 * _C,), jnp.float32),
        pltpu.VMEM((4 * _C,), jnp.float32),
        pltpu.VMEM((4 * _C,), jnp.float32),
        pltpu.VMEM((4 * _C,), jnp.float32),
        pltpu.VMEM((4 * _C,), jnp.float32),
        pltpu.VMEM((4 * _C,), jnp.float32),
        pltpu.VMEM((_LTOT,), jnp.int32),
        pltpu.VMEM((_LTOT,), jnp.float32),
        pltpu.VMEM_SHARED((_LTOT,), jnp.float32),
        pltpu.VMEM_SHARED((_LTOT,), jnp.float32),
        pltpu.VMEM((2 * _NUM_LEVELS, _C), jnp.float32),
        pltpu.SemaphoreType.DMA,
        pltpu.SemaphoreType.DMA,
    ],
)
def _encode(xc0, xc1, tab0, tab1, lidx, feats, xv0, xv1,
            ia0, ia1, wa0, wa1, ra0, ra1, rb0, rb1,
            lidxv, ltv, lsh0, lsh1, facc, sem0, sem1):
    _encode_body(xc0, xc1, tab0, tab1, lidx, feats, xv0, xv1,
                 ia0, ia1, wa0, wa1, ra0, ra1, rb0, rb1,
                 lidxv, ltv, lsh0, lsh1, facc, sem0, sem1)


_CB = 2048


def _mlp_body(f_ref, w1t_ref, b1_ref, w2t_ref, b2_ref, w3_ref, b3_ref, o_ref):
    f = f_ref[...]
    h = jnp.dot(w1t_ref[...], f, preferred_element_type=jnp.float32)
    h = jax.nn.gelu(h + b1_ref[...])
    h = jnp.dot(w2t_ref[...], h, preferred_element_type=jnp.float32)
    h = jax.nn.gelu(h + b2_ref[...])
    o = jnp.sum(h * w3_ref[...], axis=0, keepdims=True)
    o_ref[...] = o + b3_ref[...]


def _mlp(feats, w1t, b1, w2t, b2, w3, b3):
    din = 2 * _NUM_LEVELS
    grid = (_N // _CB,)
    return pl.pallas_call(
        _mlp_body,
        grid=grid,
        in_specs=[
            pl.BlockSpec((din, _CB), lambda i: (0, i)),
            pl.BlockSpec((_DH, din), lambda i: (0, 0)),
            pl.BlockSpec((_DH, 1), lambda i: (0, 0)),
            pl.BlockSpec((_DH, _DH), lambda i: (0, 0)),
            pl.BlockSpec((_DH, 1), lambda i: (0, 0)),
            pl.BlockSpec((_DH, 1), lambda i: (0, 0)),
            pl.BlockSpec((1, 1), lambda i: (0, 0)),
        ],
        out_specs=pl.BlockSpec((1, _CB), lambda i: (0, i)),
        out_shape=jax.ShapeDtypeStruct((1, _N), jnp.float32),
    )(feats, w1t, b1, w2t, b2, w3, b3)


def kernel(x, tables, W1, b1, W2, b2, W3, b3):
    xc0 = x[:, 0]
    xc1 = x[:, 1]
    tabf = tables.reshape(_NUM_LEVELS * _T, _F)
    tab0 = tabf[:, 0]
    tab1 = tabf[:, 1]
    lidx = jnp.asarray(_LIDX)
    feats = _encode(xc0, xc1, tab0, tab1, lidx)
    out = _mlp(
        feats,
        W1.T,
        b1.reshape(_DH, 1),
        W2.T,
        b2.reshape(_DH, 1),
        W3,
        b3.reshape(1, 1),
    )
    return out.reshape(_N, 1)
